# Initial kernel scaffold; baseline (speedup 1.0000x reference)
import jax
import jax.numpy as jnp
from jax.experimental import pallas as pl


def kernel(query_emb, positive_ids, negative_ids, table):
    raise NotImplementedError("write your pallas kernel here")


# speedup vs baseline: 4.0618x; 4.0618x over previous
import functools

import jax
import jax.numpy as jnp
from jax import lax
from jax.experimental import pallas as pl
from jax.experimental.pallas import tpu as pltpu
from jax.experimental.pallas import tpu_sc as plsc

_NC = 2
_NS = 16
_NW = _NC * _NS


@functools.lru_cache(maxsize=None)
def _build_gather(b_pos: int, b_neg: int, d: int):
    pos_per_w = b_pos // _NW
    neg_per_w = b_neg // _NW
    chunk = 800
    n_chunks = neg_per_w // chunk
    assert b_pos % _NW == 0 and b_neg % _NW == 0 and neg_per_w % chunk == 0
    assert pos_per_w % 8 == 0 and chunk % 8 == 0

    mesh = plsc.VectorSubcoreMesh(
        core_axis_name="c", subcore_axis_name="s",
        num_cores=_NC, num_subcores=_NS)

    @functools.partial(
        pl.kernel,
        out_type=(
            jax.ShapeDtypeStruct((b_pos, d), jnp.float32),
            jax.ShapeDtypeStruct((b_neg, d), jnp.float32),
        ),
        mesh=mesh,
        scratch_types=[
            pltpu.VMEM((pos_per_w,), jnp.int32),
            pltpu.VMEM((pos_per_w, d), jnp.float32),
            pltpu.VMEM((chunk,), jnp.int32),
            pltpu.VMEM((chunk, d), jnp.float32),
            pltpu.SemaphoreType.DMA,
        ],
        compiler_params=pltpu.CompilerParams(use_tc_tiling_on_sc=False),
    )
    def gather_k(pos_hbm, neg_hbm, table_hbm, pos_out, neg_out,
                 pidx_v, prows_v, nidx_v, nrows_v, sem):
        wid = lax.axis_index("s") * _NC + lax.axis_index("c")

        pbase = wid * pos_per_w
        pltpu.sync_copy(pos_hbm.at[pl.ds(pbase, pos_per_w)], pidx_v)
        pltpu.async_copy(table_hbm.at[pidx_v], prows_v, sem).wait()
        pltpu.sync_copy(prows_v, pos_out.at[pl.ds(pbase, pos_per_w)])

        nbase = wid * neg_per_w

        def body(j, carry):
            off = nbase + j * chunk
            pltpu.sync_copy(neg_hbm.at[pl.ds(off, chunk)], nidx_v)
            pltpu.async_copy(table_hbm.at[nidx_v], nrows_v, sem).wait()
            pltpu.sync_copy(nrows_v, neg_out.at[pl.ds(off, chunk)])
            return carry

        lax.fori_loop(0, n_chunks, body, 0)

    return gather_k


def kernel(query_emb, positive_ids, negative_ids, table):
    b, n_neg = negative_ids.shape
    _, d = table.shape
    neg_flat = negative_ids.reshape(-1)
    gather_k = _build_gather(b, b * n_neg, d)
    pos_emb, neg_emb = gather_k(positive_ids, neg_flat, table)
    return (query_emb, pos_emb, neg_emb.reshape(b, n_neg, d))


# speedup vs baseline: 4.2028x; 1.0347x over previous
import functools

import jax
import jax.numpy as jnp
from jax import lax
from jax.experimental import pallas as pl
from jax.experimental.pallas import tpu as pltpu
from jax.experimental.pallas import tpu_sc as plsc

_NC = 2
_NS = 16
_NW = _NC * _NS


@functools.lru_cache(maxsize=None)
def _build_gather(b_pos: int, b_neg: int, d: int):
    pos_per_w = b_pos // _NW
    neg_per_w = b_neg // _NW
    chunk = 800
    n_chunks = neg_per_w // chunk
    assert b_pos % _NW == 0 and b_neg % _NW == 0 and neg_per_w % chunk == 0
    assert pos_per_w % 8 == 0 and chunk % 8 == 0

    mesh = plsc.VectorSubcoreMesh(
        core_axis_name="c", subcore_axis_name="s",
        num_cores=_NC, num_subcores=_NS)

    @functools.partial(
        pl.kernel,
        out_type=(
            jax.ShapeDtypeStruct((b_pos, d), jnp.float32),
            jax.ShapeDtypeStruct((b_neg, d), jnp.float32),
        ),
        mesh=mesh,
        scratch_types=[
            pltpu.VMEM((pos_per_w,), jnp.int32),
            pltpu.VMEM((pos_per_w, d), jnp.float32),
            pltpu.VMEM((chunk,), jnp.int32),
            pltpu.VMEM((chunk,), jnp.int32),
            pltpu.VMEM((chunk, d), jnp.float32),
            pltpu.VMEM((chunk, d), jnp.float32),
            pltpu.SemaphoreType.DMA,
            pltpu.SemaphoreType.DMA,
            pltpu.SemaphoreType.DMA,
        ],
        compiler_params=pltpu.CompilerParams(use_tc_tiling_on_sc=False),
    )
    def gather_k(pos_hbm, neg_hbm, table_hbm, pos_out, neg_out,
                 pidx_v, prows_v, idx0, idx1, rows0, rows1, g0, g1, psem):
        wid = lax.axis_index("s") * _NC + lax.axis_index("c")
        idxs, rows, gsems = (idx0, idx1), (rows0, rows1), (g0, g1)

        nbase = wid * neg_per_w

        for b in range(2):
            pltpu.sync_copy(neg_hbm.at[pl.ds(nbase + b * chunk, chunk)],
                            idxs[b])
            pltpu.async_copy(table_hbm.at[idxs[b]], rows[b], gsems[b])

        pbase = wid * pos_per_w
        pltpu.sync_copy(pos_hbm.at[pl.ds(pbase, pos_per_w)], pidx_v)
        pltpu.async_copy(table_hbm.at[pidx_v], prows_v, psem).wait()
        pltpu.sync_copy(prows_v, pos_out.at[pl.ds(pbase, pos_per_w)])

        @pl.loop(0, n_chunks, step=2)
        def _ring(j):
            for b in range(2):
                cj = j + b
                pltpu.make_async_copy(table_hbm.at[idxs[b]], rows[b],
                                      gsems[b]).wait()
                pltpu.sync_copy(rows[b],
                                neg_out.at[pl.ds(nbase + cj * chunk, chunk)])
                nxt = cj + 2

                @pl.when(nxt < n_chunks)
                def _refill():
                    pltpu.sync_copy(
                        neg_hbm.at[pl.ds(nbase + nxt * chunk, chunk)], idxs[b])
                    pltpu.async_copy(table_hbm.at[idxs[b]], rows[b], gsems[b])

    return gather_k


def kernel(query_emb, positive_ids, negative_ids, table):
    b, n_neg = negative_ids.shape
    _, d = table.shape
    neg_flat = negative_ids.reshape(-1)
    gather_k = _build_gather(b, b * n_neg, d)
    pos_emb, neg_emb = gather_k(positive_ids, neg_flat, table)
    return (query_emb, pos_emb, neg_emb.reshape(b, n_neg, d))
